# Initial kernel scaffold; baseline (speedup 1.0000x reference)
import jax
import jax.numpy as jnp
from jax.experimental import pallas as pl


def kernel(x, edge_index, edge_index_cross, W1, b1, Wc1, bc1, W2, b2, Wc2, bc2, W3, b3):
    raise NotImplementedError("write your pallas kernel here")


# speedup vs baseline: 6.9536x; 6.9536x over previous
import functools

import jax
import jax.numpy as jnp
from jax import lax
from jax.experimental import pallas as pl
from jax.experimental.pallas import tpu as pltpu
from jax.experimental.pallas import tpu_sc as plsc

N = 10000
E = 160000
D = 256
DH = 128
NC = 2
NS = 16
EPT = E // NS
CH = 80
NCH = EPT // CH
RCH = 400
NRC = N // RCH

_SC_MESH = plsc.VectorSubcoreMesh(
    core_axis_name="c", subcore_axis_name="s", num_cores=NC, num_subcores=NS)


def _sc_scatter_body(xws, src, dst, out, acc, rows, sraw, gidx, sidx):
    c = lax.axis_index("c")
    s = lax.axis_index("s")
    half0 = c * N

    for t in range(2):
        cid = s + NS * t

        @pl.when(cid < NRC)
        def _():
            r = cid * RCH
            pltpu.sync_copy(xws.at[pl.ds(half0 + r, RCH)], acc.at[pl.ds(r, RCH)])

    plsc.subcore_barrier()

    def step(k, carry):
        base = s * EPT + k * CH
        pltpu.sync_copy(src.at[pl.ds(base, CH)], sraw)
        pltpu.sync_copy(dst.at[pl.ds(base, CH)], sidx.at[0])
        for t in range(CH // 16):
            gidx[pl.ds(t * 16, 16)] = sraw[pl.ds(t * 16, 16)] + half0
        pltpu.sync_copy(xws.at[gidx], rows)
        pltpu.sync_copy(rows, acc.at[sidx.at[0]], add=True)
        return carry

    lax.fori_loop(0, NCH, step, 0)
    plsc.subcore_barrier()

    for t in range(2):
        cid = s + NS * t

        @pl.when(cid < NRC)
        def _():
            r = cid * RCH
            pltpu.sync_copy(acc.at[pl.ds(r, RCH)], out.at[pl.ds(half0 + r, RCH)])


_sc_scatter = pl.kernel(
    _sc_scatter_body,
    out_type=jax.ShapeDtypeStruct((NC * N, DH), jnp.float32),
    mesh=_SC_MESH,
    scratch_types=[
        pltpu.VMEM_SHARED((N, DH), jnp.float32),
        pltpu.VMEM((CH, DH), jnp.float32),
        pltpu.VMEM((CH,), jnp.int32),
        pltpu.VMEM((CH,), jnp.int32),
        pltpu.VMEM((1, CH), jnp.int32),
    ],
)


def _sc_degrees_body(dsta, dstc, dega, degc, sh, buf, sidx):
    c = lax.axis_index("c")
    s = lax.axis_index("s")

    def fill(i, carry):
        buf[i, :] = jnp.ones((16,), jnp.float32)
        return carry

    lax.fori_loop(0, RCH, fill, 0)
    for t in range(2):
        cid = s + NS * t

        @pl.when(cid < NRC)
        def _():
            pltpu.sync_copy(buf, sh.at[pl.ds(cid * RCH, RCH)])

    plsc.subcore_barrier()

    def step(k, carry):
        base = s * EPT + k * CH

        @pl.when(c == 0)
        def _():
            pltpu.sync_copy(dsta.at[pl.ds(base, CH)], sidx.at[0])

        @pl.when(c == 1)
        def _():
            pltpu.sync_copy(dstc.at[pl.ds(base, CH)], sidx.at[0])

        pltpu.sync_copy(buf.at[pl.ds(0, CH)], sh.at[sidx.at[0]], add=True)
        return carry

    lax.fori_loop(0, NCH, step, 0)
    plsc.subcore_barrier()

    for t in range(2):
        cid = s + NS * t

        @pl.when(cid < NRC)
        def _():
            pltpu.sync_copy(sh.at[pl.ds(cid * RCH, RCH)], buf)

            @pl.when(c == 0)
            def _():
                pltpu.sync_copy(buf, dega.at[pl.ds(cid * RCH, RCH)])

            @pl.when(c == 1)
            def _():
                pltpu.sync_copy(buf, degc.at[pl.ds(cid * RCH, RCH)])


_sc_degrees = pl.kernel(
    _sc_degrees_body,
    out_type=(jax.ShapeDtypeStruct((N, 16), jnp.float32),
              jax.ShapeDtypeStruct((N, 16), jnp.float32)),
    mesh=_SC_MESH,
    scratch_types=[
        pltpu.VMEM_SHARED((N, 16), jnp.float32),
        pltpu.VMEM((RCH, 16), jnp.float32),
        pltpu.VMEM((1, CH), jnp.int32),
    ],
)


BR = 400
GB = N // BR


def _mm_first_body(x_ref, w_ref, degn_ref, out_ref):
    xw = jnp.dot(x_ref[...], w_ref[...], preferred_element_type=jnp.float32)
    sc = lax.rsqrt(degn_ref[...])
    out_ref[0] = xw[:, :DH] * sc
    out_ref[1] = xw[:, DH:] * sc


_mm_first = pl.pallas_call(
    _mm_first_body,
    grid=(GB,),
    in_specs=[
        pl.BlockSpec((BR, D), lambda i: (i, 0)),
        pl.BlockSpec((D, D), lambda i: (0, 0)),
        pl.BlockSpec((BR, 1), lambda i: (i, 0)),
    ],
    out_specs=pl.BlockSpec((2, BR, DH), lambda i: (0, i, 0)),
    out_shape=jax.ShapeDtypeStruct((2, N, DH), jnp.float32),
)


def _mm_mid_body(agg_ref, degp_ref, bp_ref, w_ref, degn_ref, out_ref):
    dp = lax.rsqrt(degp_ref[...])
    h0 = jnp.maximum(agg_ref[0] * dp + bp_ref[:, :DH], 0.0)
    h1 = jnp.maximum(agg_ref[1] * dp + bp_ref[:, DH:], 0.0)
    xw = (jnp.dot(h0, w_ref[:DH, :], preferred_element_type=jnp.float32)
          + jnp.dot(h1, w_ref[DH:, :], preferred_element_type=jnp.float32))
    sc = lax.rsqrt(degn_ref[...])
    out_ref[0] = xw[:, :DH] * sc
    out_ref[1] = xw[:, DH:] * sc


_mm_mid = pl.pallas_call(
    _mm_mid_body,
    grid=(GB,),
    in_specs=[
        pl.BlockSpec((2, BR, DH), lambda i: (0, i, 0)),
        pl.BlockSpec((BR, 1), lambda i: (i, 0)),
        pl.BlockSpec((1, D), lambda i: (0, 0)),
        pl.BlockSpec((D, D), lambda i: (0, 0)),
        pl.BlockSpec((BR, 1), lambda i: (i, 0)),
    ],
    out_specs=pl.BlockSpec((2, BR, DH), lambda i: (0, i, 0)),
    out_shape=jax.ShapeDtypeStruct((2, N, DH), jnp.float32),
)


def _fin_body(agg_ref, deg_ref, b_ref, out_ref):
    dp = lax.rsqrt(deg_ref[...])
    out_ref[:, :DH] = agg_ref[0] * dp + b_ref[:, :DH]
    out_ref[:, DH:] = agg_ref[1] * dp + b_ref[:, DH:]


_finalize = pl.pallas_call(
    _fin_body,
    grid=(GB,),
    in_specs=[
        pl.BlockSpec((2, BR, DH), lambda i: (0, i, 0)),
        pl.BlockSpec((BR, 1), lambda i: (i, 0)),
        pl.BlockSpec((1, D), lambda i: (0, 0)),
    ],
    out_specs=pl.BlockSpec((BR, D), lambda i: (i, 0)),
    out_shape=jax.ShapeDtypeStruct((N, D), jnp.float32),
)


def _agg(xws, src, dst):
    return _sc_scatter(xws.reshape(NC * N, DH), src, dst).reshape(NC, N, DH)


def kernel(x, edge_index, edge_index_cross, W1, b1, Wc1, bc1, W2, b2, Wc2, bc2, W3, b3):
    src_a = edge_index[0].astype(jnp.int32)
    dst_a = edge_index[1].astype(jnp.int32)
    src_c = edge_index_cross[0].astype(jnp.int32)
    dst_c = edge_index_cross[1].astype(jnp.int32)

    dega16, degc16 = _sc_degrees(dst_a, dst_c)
    deg_a = dega16[:, :1]
    deg_c = degc16[:, :1]

    b1r, bc1r, b2r, bc2r, b3r = (b.reshape(1, D) for b in (b1, bc1, b2, bc2, b3))

    xws = _mm_first(x, W1, deg_a)
    agg = _agg(xws, src_a, dst_a)
    xws = _mm_mid(agg, deg_a, b1r, Wc1, deg_c)
    agg = _agg(xws, src_c, dst_c)
    xws = _mm_mid(agg, deg_c, bc1r, W2, deg_a)
    agg = _agg(xws, src_a, dst_a)
    xws = _mm_mid(agg, deg_a, b2r, Wc2, deg_c)
    agg = _agg(xws, src_c, dst_c)
    xws = _mm_mid(agg, deg_c, bc2r, W3, deg_a)
    agg = _agg(xws, src_a, dst_a)
    return _finalize(agg, deg_a, b3r)


# speedup vs baseline: 14.9494x; 2.1499x over previous
import functools

import jax
import jax.numpy as jnp
from jax import lax
from jax.experimental import pallas as pl
from jax.experimental.pallas import tpu as pltpu
from jax.experimental.pallas import tpu_sc as plsc

N = 10000
E = 160000
D = 256
DH = 128
NC = 2
NS = 16
EPT = E // NS
CH = 80
NCH = EPT // CH
RCH = 400
NRC = N // RCH

_SC_MESH = plsc.VectorSubcoreMesh(
    core_axis_name="c", subcore_axis_name="s", num_cores=NC, num_subcores=NS)


def _sc_scatter_body(xws, src, dst3, out, acc, rows, srcv, dstv, sem):
    c = lax.axis_index("c")
    s = lax.axis_index("s")
    half0 = c * N

    pltpu.sync_copy(src.at[pl.ds(s * EPT, EPT)], srcv)
    pltpu.sync_copy(dst3.at[s], dstv)

    for t in range(2):
        cid = s + NS * t

        @pl.when(cid < NRC)
        def _():
            r = cid * RCH
            pltpu.sync_copy(xws.at[pl.ds(half0 + r, RCH)], acc.at[pl.ds(r, RCH)])

    plsc.subcore_barrier()

    tab = xws.at[pl.ds(half0, N)]

    pltpu.async_copy(tab.at[srcv.at[pl.ds(0, CH)]], rows.at[0], sem)

    def step(k, carry):
        j = lax.rem(k, 2)

        @pl.when(k + 1 < NCH)
        def _():
            pltpu.async_copy(
                tab.at[srcv.at[pl.ds((k + 1) * CH, CH)]], rows.at[1 - j], sem)

        pltpu.make_async_copy(xws.at[pl.ds(0, CH)], rows.at[j], sem).wait()
        pltpu.sync_copy(rows.at[j], acc.at[dstv.at[k]], add=True)
        return carry

    lax.fori_loop(0, NCH, step, 0)
    plsc.subcore_barrier()

    for t in range(2):
        cid = s + NS * t

        @pl.when(cid < NRC)
        def _():
            r = cid * RCH
            pltpu.sync_copy(acc.at[pl.ds(r, RCH)], out.at[pl.ds(half0 + r, RCH)])


_sc_scatter = pl.kernel(
    _sc_scatter_body,
    out_type=jax.ShapeDtypeStruct((NC * N, DH), jnp.float32),
    mesh=_SC_MESH,
    scratch_types=[
        pltpu.VMEM_SHARED((N, DH), jnp.float32),
        pltpu.VMEM((2, CH, DH), jnp.float32),
        pltpu.VMEM((EPT,), jnp.int32),
        pltpu.VMEM((NCH, CH), jnp.int32),
        pltpu.SemaphoreType.DMA,
    ],
)


def _sc_degrees_body(dsta, dstc, dega, degc, sh, buf, sidx):
    c = lax.axis_index("c")
    s = lax.axis_index("s")

    def fill(i, carry):
        buf[i, :] = jnp.ones((16,), jnp.float32)
        return carry

    lax.fori_loop(0, RCH, fill, 0)
    for t in range(2):
        cid = s + NS * t

        @pl.when(cid < NRC)
        def _():
            pltpu.sync_copy(buf, sh.at[pl.ds(cid * RCH, RCH)])

    plsc.subcore_barrier()

    def step(k, carry):
        base = s * EPT + k * CH

        @pl.when(c == 0)
        def _():
            pltpu.sync_copy(dsta.at[pl.ds(base, CH)], sidx.at[0])

        @pl.when(c == 1)
        def _():
            pltpu.sync_copy(dstc.at[pl.ds(base, CH)], sidx.at[0])

        pltpu.sync_copy(buf.at[pl.ds(0, CH)], sh.at[sidx.at[0]], add=True)
        return carry

    lax.fori_loop(0, NCH, step, 0)
    plsc.subcore_barrier()

    for t in range(2):
        cid = s + NS * t

        @pl.when(cid < NRC)
        def _():
            pltpu.sync_copy(sh.at[pl.ds(cid * RCH, RCH)], buf)

            @pl.when(c == 0)
            def _():
                pltpu.sync_copy(buf, dega.at[pl.ds(cid * RCH, RCH)])

            @pl.when(c == 1)
            def _():
                pltpu.sync_copy(buf, degc.at[pl.ds(cid * RCH, RCH)])


_sc_degrees = pl.kernel(
    _sc_degrees_body,
    out_type=(jax.ShapeDtypeStruct((N, 16), jnp.float32),
              jax.ShapeDtypeStruct((N, 16), jnp.float32)),
    mesh=_SC_MESH,
    scratch_types=[
        pltpu.VMEM_SHARED((N, 16), jnp.float32),
        pltpu.VMEM((RCH, 16), jnp.float32),
        pltpu.VMEM((1, CH), jnp.int32),
    ],
)


BR = 400
GB = N // BR


def _mm_first_body(x_ref, w_ref, degn_ref, out_ref):
    xw = jnp.dot(x_ref[...], w_ref[...], preferred_element_type=jnp.float32)
    sc = lax.rsqrt(degn_ref[...])
    out_ref[0] = xw[:, :DH] * sc
    out_ref[1] = xw[:, DH:] * sc


_mm_first = pl.pallas_call(
    _mm_first_body,
    grid=(GB,),
    in_specs=[
        pl.BlockSpec((BR, D), lambda i: (i, 0)),
        pl.BlockSpec((D, D), lambda i: (0, 0)),
        pl.BlockSpec((BR, 1), lambda i: (i, 0)),
    ],
    out_specs=pl.BlockSpec((2, BR, DH), lambda i: (0, i, 0)),
    out_shape=jax.ShapeDtypeStruct((2, N, DH), jnp.float32),
)


def _mm_mid_body(agg_ref, degp_ref, bp_ref, w_ref, degn_ref, out_ref):
    dp = lax.rsqrt(degp_ref[...])
    h0 = jnp.maximum(agg_ref[0] * dp + bp_ref[:, :DH], 0.0)
    h1 = jnp.maximum(agg_ref[1] * dp + bp_ref[:, DH:], 0.0)
    xw = (jnp.dot(h0, w_ref[:DH, :], preferred_element_type=jnp.float32)
          + jnp.dot(h1, w_ref[DH:, :], preferred_element_type=jnp.float32))
    sc = lax.rsqrt(degn_ref[...])
    out_ref[0] = xw[:, :DH] * sc
    out_ref[1] = xw[:, DH:] * sc


_mm_mid = pl.pallas_call(
    _mm_mid_body,
    grid=(GB,),
    in_specs=[
        pl.BlockSpec((2, BR, DH), lambda i: (0, i, 0)),
        pl.BlockSpec((BR, 1), lambda i: (i, 0)),
        pl.BlockSpec((1, D), lambda i: (0, 0)),
        pl.BlockSpec((D, D), lambda i: (0, 0)),
        pl.BlockSpec((BR, 1), lambda i: (i, 0)),
    ],
    out_specs=pl.BlockSpec((2, BR, DH), lambda i: (0, i, 0)),
    out_shape=jax.ShapeDtypeStruct((2, N, DH), jnp.float32),
)


def _fin_body(agg_ref, deg_ref, b_ref, out_ref):
    dp = lax.rsqrt(deg_ref[...])
    out_ref[:, :DH] = agg_ref[0] * dp + b_ref[:, :DH]
    out_ref[:, DH:] = agg_ref[1] * dp + b_ref[:, DH:]


_finalize = pl.pallas_call(
    _fin_body,
    grid=(GB,),
    in_specs=[
        pl.BlockSpec((2, BR, DH), lambda i: (0, i, 0)),
        pl.BlockSpec((BR, 1), lambda i: (i, 0)),
        pl.BlockSpec((1, D), lambda i: (0, 0)),
    ],
    out_specs=pl.BlockSpec((BR, D), lambda i: (i, 0)),
    out_shape=jax.ShapeDtypeStruct((N, D), jnp.float32),
)


def _agg(xws, src, dst):
    return _sc_scatter(
        xws.reshape(NC * N, DH), src, dst.reshape(NS, NCH, CH)).reshape(NC, N, DH)


def kernel(x, edge_index, edge_index_cross, W1, b1, Wc1, bc1, W2, b2, Wc2, bc2, W3, b3):
    src_a = edge_index[0].astype(jnp.int32)
    dst_a = edge_index[1].astype(jnp.int32)
    src_c = edge_index_cross[0].astype(jnp.int32)
    dst_c = edge_index_cross[1].astype(jnp.int32)

    dega16, degc16 = _sc_degrees(dst_a, dst_c)
    deg_a = dega16[:, :1]
    deg_c = degc16[:, :1]

    b1r, bc1r, b2r, bc2r, b3r = (b.reshape(1, D) for b in (b1, bc1, b2, bc2, b3))

    xws = _mm_first(x, W1, deg_a)
    agg = _agg(xws, src_a, dst_a)
    xws = _mm_mid(agg, deg_a, b1r, Wc1, deg_c)
    agg = _agg(xws, src_c, dst_c)
    xws = _mm_mid(agg, deg_c, bc1r, W2, deg_a)
    agg = _agg(xws, src_a, dst_a)
    xws = _mm_mid(agg, deg_a, b2r, Wc2, deg_c)
    agg = _agg(xws, src_c, dst_c)
    xws = _mm_mid(agg, deg_c, bc2r, W3, deg_a)
    agg = _agg(xws, src_a, dst_a)
    return _finalize(agg, deg_a, b3r)


# speedup vs baseline: 16.1083x; 1.0775x over previous
import functools

import jax
import jax.numpy as jnp
from jax import lax
from jax.experimental import pallas as pl
from jax.experimental.pallas import tpu as pltpu
from jax.experimental.pallas import tpu_sc as plsc

N = 10000
E = 160000
D = 256
DH = 128
NC = 2
NS = 16
EPT = E // NS
CH = 80
NCH = EPT // CH
RCH = 400
NRC = N // RCH

_SC_MESH = plsc.VectorSubcoreMesh(
    core_axis_name="c", subcore_axis_name="s", num_cores=NC, num_subcores=NS)


def _sc_scatter_body(xws, src, dst3, out, acc, rows, srcv, dstv, gsem):
    c = lax.axis_index("c")
    s = lax.axis_index("s")
    half0 = c * N

    pltpu.sync_copy(src.at[pl.ds(s * EPT, EPT)], srcv)
    pltpu.sync_copy(dst3.at[s], dstv)

    for t in range(2):
        cid = s + NS * t

        @pl.when(cid < NRC)
        def _():
            r = cid * RCH
            pltpu.sync_copy(xws.at[pl.ds(half0 + r, RCH)], acc.at[pl.ds(r, RCH)])

    plsc.subcore_barrier()

    tab = xws.at[pl.ds(half0, N)]

    pltpu.async_copy(tab.at[srcv.at[pl.ds(0, CH)]], rows.at[0], gsem)

    def step(k, carry):
        j = lax.rem(k, 2)

        @pl.when(k + 1 < NCH)
        def _():
            pltpu.async_copy(
                tab.at[srcv.at[pl.ds((k + 1) * CH, CH)]], rows.at[1 - j], gsem)

        pltpu.make_async_copy(xws.at[pl.ds(0, CH)], rows.at[j], gsem).wait()
        pltpu.sync_copy(rows.at[j], acc.at[dstv.at[k]], add=True)
        return carry

    lax.fori_loop(0, NCH, step, 0)
    plsc.subcore_barrier()

    for t in range(2):
        cid = s + NS * t

        @pl.when(cid < NRC)
        def _():
            r = cid * RCH
            pltpu.sync_copy(acc.at[pl.ds(r, RCH)], out.at[pl.ds(half0 + r, RCH)])


_sc_scatter = pl.kernel(
    _sc_scatter_body,
    out_type=jax.ShapeDtypeStruct((NC * N, DH), jnp.float32),
    mesh=_SC_MESH,
    scratch_types=[
        pltpu.VMEM_SHARED((N, DH), jnp.float32),
        pltpu.VMEM((2, CH, DH), jnp.float32),
        pltpu.VMEM((EPT,), jnp.int32),
        pltpu.VMEM((NCH, CH), jnp.int32),
        pltpu.SemaphoreType.DMA,
    ],
)


def _sc_degrees_body(dsta, dstc, dega, degc, sh, buf, sidx):
    c = lax.axis_index("c")
    s = lax.axis_index("s")

    def fill(i, carry):
        buf[i, :] = jnp.ones((16,), jnp.float32)
        return carry

    lax.fori_loop(0, RCH, fill, 0)
    for t in range(2):
        cid = s + NS * t

        @pl.when(cid < NRC)
        def _():
            pltpu.sync_copy(buf, sh.at[pl.ds(cid * RCH, RCH)])

    plsc.subcore_barrier()

    def step(k, carry):
        base = s * EPT + k * CH

        @pl.when(c == 0)
        def _():
            pltpu.sync_copy(dsta.at[pl.ds(base, CH)], sidx.at[0])

        @pl.when(c == 1)
        def _():
            pltpu.sync_copy(dstc.at[pl.ds(base, CH)], sidx.at[0])

        pltpu.sync_copy(buf.at[pl.ds(0, CH)], sh.at[sidx.at[0]], add=True)
        return carry

    lax.fori_loop(0, NCH, step, 0)
    plsc.subcore_barrier()

    for t in range(2):
        cid = s + NS * t

        @pl.when(cid < NRC)
        def _():
            pltpu.sync_copy(sh.at[pl.ds(cid * RCH, RCH)], buf)

            @pl.when(c == 0)
            def _():
                pltpu.sync_copy(buf, dega.at[pl.ds(cid * RCH, RCH)])

            @pl.when(c == 1)
            def _():
                pltpu.sync_copy(buf, degc.at[pl.ds(cid * RCH, RCH)])


_sc_degrees = pl.kernel(
    _sc_degrees_body,
    out_type=(jax.ShapeDtypeStruct((N, 16), jnp.float32),
              jax.ShapeDtypeStruct((N, 16), jnp.float32)),
    mesh=_SC_MESH,
    scratch_types=[
        pltpu.VMEM_SHARED((N, 16), jnp.float32),
        pltpu.VMEM((RCH, 16), jnp.float32),
        pltpu.VMEM((1, CH), jnp.int32),
    ],
)


BR = 2000
GB = N // BR


def _mm_first_body(x_ref, w_ref, degn_ref, out_ref):
    xw = jnp.dot(x_ref[...], w_ref[...], preferred_element_type=jnp.float32)
    sc = lax.rsqrt(degn_ref[...])
    out_ref[0] = xw[:, :DH] * sc
    out_ref[1] = xw[:, DH:] * sc


_mm_first = pl.pallas_call(
    _mm_first_body,
    grid=(GB,),
    in_specs=[
        pl.BlockSpec((BR, D), lambda i: (i, 0)),
        pl.BlockSpec((D, D), lambda i: (0, 0)),
        pl.BlockSpec((BR, 1), lambda i: (i, 0)),
    ],
    out_specs=pl.BlockSpec((2, BR, DH), lambda i: (0, i, 0)),
    out_shape=jax.ShapeDtypeStruct((2, N, DH), jnp.float32),
)


def _mm_mid_body(agg_ref, degp_ref, bp_ref, w_ref, degn_ref, out_ref):
    dp = lax.rsqrt(degp_ref[...])
    h0 = jnp.maximum(agg_ref[0] * dp + bp_ref[:, :DH], 0.0)
    h1 = jnp.maximum(agg_ref[1] * dp + bp_ref[:, DH:], 0.0)
    xw = (jnp.dot(h0, w_ref[:DH, :], preferred_element_type=jnp.float32)
          + jnp.dot(h1, w_ref[DH:, :], preferred_element_type=jnp.float32))
    sc = lax.rsqrt(degn_ref[...])
    out_ref[0] = xw[:, :DH] * sc
    out_ref[1] = xw[:, DH:] * sc


_mm_mid = pl.pallas_call(
    _mm_mid_body,
    grid=(GB,),
    in_specs=[
        pl.BlockSpec((2, BR, DH), lambda i: (0, i, 0)),
        pl.BlockSpec((BR, 1), lambda i: (i, 0)),
        pl.BlockSpec((1, D), lambda i: (0, 0)),
        pl.BlockSpec((D, D), lambda i: (0, 0)),
        pl.BlockSpec((BR, 1), lambda i: (i, 0)),
    ],
    out_specs=pl.BlockSpec((2, BR, DH), lambda i: (0, i, 0)),
    out_shape=jax.ShapeDtypeStruct((2, N, DH), jnp.float32),
)


def _fin_body(agg_ref, deg_ref, b_ref, out_ref):
    dp = lax.rsqrt(deg_ref[...])
    out_ref[:, :DH] = agg_ref[0] * dp + b_ref[:, :DH]
    out_ref[:, DH:] = agg_ref[1] * dp + b_ref[:, DH:]


_finalize = pl.pallas_call(
    _fin_body,
    grid=(GB,),
    in_specs=[
        pl.BlockSpec((2, BR, DH), lambda i: (0, i, 0)),
        pl.BlockSpec((BR, 1), lambda i: (i, 0)),
        pl.BlockSpec((1, D), lambda i: (0, 0)),
    ],
    out_specs=pl.BlockSpec((BR, D), lambda i: (i, 0)),
    out_shape=jax.ShapeDtypeStruct((N, D), jnp.float32),
)


def _agg(xws, src, dst):
    return _sc_scatter(
        xws.reshape(NC * N, DH), src, dst.reshape(NS, NCH, CH)).reshape(NC, N, DH)


def kernel(x, edge_index, edge_index_cross, W1, b1, Wc1, bc1, W2, b2, Wc2, bc2, W3, b3):
    src_a = edge_index[0].astype(jnp.int32)
    dst_a = edge_index[1].astype(jnp.int32)
    src_c = edge_index_cross[0].astype(jnp.int32)
    dst_c = edge_index_cross[1].astype(jnp.int32)

    dega16, degc16 = _sc_degrees(dst_a, dst_c)
    deg_a = dega16[:, :1]
    deg_c = degc16[:, :1]

    b1r, bc1r, b2r, bc2r, b3r = (b.reshape(1, D) for b in (b1, bc1, b2, bc2, b3))

    xws = _mm_first(x, W1, deg_a)
    agg = _agg(xws, src_a, dst_a)
    xws = _mm_mid(agg, deg_a, b1r, Wc1, deg_c)
    agg = _agg(xws, src_c, dst_c)
    xws = _mm_mid(agg, deg_c, bc1r, W2, deg_a)
    agg = _agg(xws, src_a, dst_a)
    xws = _mm_mid(agg, deg_a, b2r, Wc2, deg_c)
    agg = _agg(xws, src_c, dst_c)
    xws = _mm_mid(agg, deg_c, bc2r, W3, deg_a)
    agg = _agg(xws, src_a, dst_a)
    return _finalize(agg, deg_a, b3r)
